# baseline (speedup 1.0000x reference)
import jax
import jax.numpy as jnp
from jax import lax
from jax.experimental import pallas as pl
from jax.experimental.pallas import tpu as pltpu

H = 384
W = 384
C = 192
NPOS = 31
NSEG = NPOS + 1
HB = 32


def _reduce_kernel(hp_ref, vp_ref, x_ref, colexp_ref, ridx_ref, s1_ref):
    h = pl.program_id(0)
    nsteps = pl.num_programs(0)
    col_i = lax.broadcasted_iota(jnp.int32, (HB, 1), 0) + h * HB
    row_i = lax.broadcasted_iota(jnp.int32, (1, HB), 1) + h * HB
    acc_c = jnp.zeros((HB, 1), jnp.int32)
    acc_r = jnp.zeros((1, HB), jnp.int32)
    for k in range(NPOS):
        p = hp_ref[0, k]
        acc_c += (p <= col_i).astype(jnp.int32)
        acc_r += (p <= row_i).astype(jnp.int32)
    ridx_ref[...] = acc_c
    onehot_t = (acc_r == lax.broadcasted_iota(jnp.int32, (NSEG, HB), 0)
                ).astype(jnp.float32)
    part = lax.dot_general(onehot_t, x_ref[...], (((1,), (0,)), ((), ())),
                           preferred_element_type=jnp.float32)

    @pl.when(h == 0)
    def _():
        s1_ref[...] = part

    @pl.when(h > 0)
    def _():
        s1_ref[...] += part

    @pl.when(h == nsteps - 1)
    def _():
        jj_r = lax.broadcasted_iota(jnp.int32, (1, W), 1)
        jj_c = lax.broadcasted_iota(jnp.int32, (W, 1), 0)
        acc_jr = jnp.zeros((1, W), jnp.int32)
        acc_jc = jnp.zeros((W, 1), jnp.int32)
        for k in range(NPOS):
            p = vp_ref[0, k]
            acc_jr += (p <= jj_r).astype(jnp.int32)
            acc_jc += (p <= jj_c).astype(jnp.int32)
        ohct = (acc_jr == lax.broadcasted_iota(jnp.int32, (NSEG, W), 0)
                ).astype(jnp.float32)
        ohc = (acc_jc == lax.broadcasted_iota(jnp.int32, (W, NSEG), 1)
               ).astype(jnp.float32)
        cnt = jnp.sum(ohc, axis=0, keepdims=True)
        ohc_s = ohc * (1.0 / jnp.maximum(cnt, 1.0))
        for r in range(NSEG):
            lo = hp_ref[0, r - 1] if r > 0 else 0
            hi = hp_ref[0, r] if r < NPOS else H
            rs = 1.0 / jnp.maximum(hi - lo, 1).astype(jnp.float32)
            mean_r = lax.dot_general(
                s1_ref[r], ohc_s, (((1,), (0,)), ((), ())),
                preferred_element_type=jnp.float32)
            colexp_ref[r] = lax.dot_general(
                mean_r * rs, ohct, (((1,), (0,)), ((), ())),
                preferred_element_type=jnp.float32)


def _row_gather_kernel(ridx_ref, colexp_ref, out_ref):
    out_ref[...] = colexp_ref[...]


def kernel(input, h_positions, v_positions):
    xt = jnp.transpose(input[0], (0, 2, 1))
    hp = h_positions.astype(jnp.int32).reshape(1, NPOS)
    vp = v_positions.astype(jnp.int32).reshape(1, NPOS)

    colexp, ridx = pl.pallas_call(
        _reduce_kernel,
        grid=(H // HB,),
        in_specs=[
            pl.BlockSpec(memory_space=pltpu.SMEM),
            pl.BlockSpec(memory_space=pltpu.SMEM),
            pl.BlockSpec((HB, C, W), lambda h: (h, 0, 0)),
        ],
        out_specs=[
            pl.BlockSpec((NSEG, C, W), lambda h: (0, 0, 0)),
            pl.BlockSpec((HB, 1), lambda h: (h, 0)),
        ],
        out_shape=[
            jax.ShapeDtypeStruct((NSEG, C, W), jnp.float32),
            jax.ShapeDtypeStruct((H, 1), jnp.int32),
        ],
        scratch_shapes=[pltpu.VMEM((NSEG, C, W), jnp.float32)],
    )(hp, vp, xt)

    yt = pl.pallas_call(
        _row_gather_kernel,
        grid_spec=pltpu.PrefetchScalarGridSpec(
            num_scalar_prefetch=1,
            grid=(H,),
            in_specs=[
                pl.BlockSpec((1, C, W), lambda i, ridx: (ridx[i], 0, 0)),
            ],
            out_specs=pl.BlockSpec((1, C, W), lambda i, ridx: (i, 0, 0)),
        ),
        out_shape=jax.ShapeDtypeStruct((H, C, W), jnp.float32),
    )(ridx.reshape(H), colexp)

    return jnp.transpose(yt, (0, 2, 1))[None]


# speedup vs baseline: 1.9702x; 1.9702x over previous
import jax
import jax.numpy as jnp
from jax import lax
from jax.experimental import pallas as pl
from jax.experimental.pallas import tpu as pltpu

H = 384
W = 384
C = 192
NPOS = 31
NSEG = NPOS + 1
HB = 32


def _reduce_kernel(hp_ref, vp_ref, x_ref, colexp_ref, ridx_ref, s1_ref):
    h = pl.program_id(0)
    nsteps = pl.num_programs(0)
    col_i = lax.broadcasted_iota(jnp.int32, (HB, 1), 0) + h * HB
    row_i = lax.broadcasted_iota(jnp.int32, (1, HB), 1) + h * HB
    acc_c = jnp.zeros((HB, 1), jnp.int32)
    acc_r = jnp.zeros((1, HB), jnp.int32)
    for k in range(NPOS):
        p = hp_ref[0, k]
        acc_c += (p <= col_i).astype(jnp.int32)
        acc_r += (p <= row_i).astype(jnp.int32)
    ridx_ref[...] = acc_c
    onehot_t = (acc_r == lax.broadcasted_iota(jnp.int32, (NSEG, HB), 0)
                ).astype(jnp.float32)
    part = lax.dot_general(onehot_t, x_ref[...], (((1,), (0,)), ((), ())),
                           preferred_element_type=jnp.float32)

    @pl.when(h == 0)
    def _():
        s1_ref[...] = part

    @pl.when(h > 0)
    def _():
        s1_ref[...] += part

    @pl.when(h == nsteps - 1)
    def _():
        jj_r = lax.broadcasted_iota(jnp.int32, (1, W), 1)
        jj_c = lax.broadcasted_iota(jnp.int32, (W, 1), 0)
        acc_jr = jnp.zeros((1, W), jnp.int32)
        acc_jc = jnp.zeros((W, 1), jnp.int32)
        for k in range(NPOS):
            p = vp_ref[0, k]
            acc_jr += (p <= jj_r).astype(jnp.int32)
            acc_jc += (p <= jj_c).astype(jnp.int32)
        ohct = (acc_jr == lax.broadcasted_iota(jnp.int32, (NSEG, W), 0)
                ).astype(jnp.float32)
        ohc = (acc_jc == lax.broadcasted_iota(jnp.int32, (W, NSEG), 1)
               ).astype(jnp.float32)
        cnt = jnp.sum(ohc, axis=0, keepdims=True)
        ohc_s = ohc * (1.0 / jnp.maximum(cnt, 1.0))
        for r in range(NSEG):
            lo = hp_ref[0, r - 1] if r > 0 else 0
            hi = hp_ref[0, r] if r < NPOS else H
            rs = 1.0 / jnp.maximum(hi - lo, 1).astype(jnp.float32)
            mean_r = lax.dot_general(
                s1_ref[r], ohc_s, (((1,), (0,)), ((), ())),
                preferred_element_type=jnp.float32)
            colexp_ref[r] = lax.dot_general(
                mean_r * rs, ohct, (((1,), (0,)), ((), ())),
                preferred_element_type=jnp.float32)


IB = 32


def _row_gather_kernel(ridx_ref, colexp_ref, out_ref):
    base = pl.program_id(0) * IB

    def body(ii, carry):
        r = ridx_ref[base + ii]
        out_ref[pl.ds(ii, 1)] = colexp_ref[pl.ds(r, 1)]
        return carry

    lax.fori_loop(0, IB, body, 0)


def kernel(input, h_positions, v_positions):
    xt = jnp.transpose(input[0], (0, 2, 1))
    hp = h_positions.astype(jnp.int32).reshape(1, NPOS)
    vp = v_positions.astype(jnp.int32).reshape(1, NPOS)

    colexp, ridx = pl.pallas_call(
        _reduce_kernel,
        grid=(H // HB,),
        in_specs=[
            pl.BlockSpec(memory_space=pltpu.SMEM),
            pl.BlockSpec(memory_space=pltpu.SMEM),
            pl.BlockSpec((HB, C, W), lambda h: (h, 0, 0)),
        ],
        out_specs=[
            pl.BlockSpec((NSEG, C, W), lambda h: (0, 0, 0)),
            pl.BlockSpec((HB, 1), lambda h: (h, 0)),
        ],
        out_shape=[
            jax.ShapeDtypeStruct((NSEG, C, W), jnp.float32),
            jax.ShapeDtypeStruct((H, 1), jnp.int32),
        ],
        scratch_shapes=[pltpu.VMEM((NSEG, C, W), jnp.float32)],
    )(hp, vp, xt)

    yt = pl.pallas_call(
        _row_gather_kernel,
        grid=(H // IB,),
        in_specs=[
            pl.BlockSpec(memory_space=pltpu.SMEM),
            pl.BlockSpec((NSEG, C, W), lambda h: (0, 0, 0)),
        ],
        out_specs=pl.BlockSpec((IB, C, W), lambda h: (h, 0, 0)),
        out_shape=jax.ShapeDtypeStruct((H, C, W), jnp.float32),
    )(ridx.reshape(H), colexp)

    return jnp.transpose(yt, (0, 2, 1))[None]


# speedup vs baseline: 2.3447x; 1.1900x over previous
import jax
import jax.numpy as jnp
from jax import lax
from jax.experimental import pallas as pl
from jax.experimental.pallas import tpu as pltpu

H = 384
W = 384
C = 192
NPOS = 31
NSEG = NPOS + 1
HB = 64
IB = 32


def _reduce_kernel(hp_ref, vp_ref, x_ref, colexp_ref, ridx_ref):
    h = pl.program_id(0)
    nsteps = pl.num_programs(0)
    col_i = lax.broadcasted_iota(jnp.int32, (HB, 1), 0) + h * HB
    row_i = lax.broadcasted_iota(jnp.int32, (1, HB), 1) + h * HB
    acc_c = jnp.zeros((HB, 1), jnp.int32)
    acc_r = jnp.zeros((1, HB), jnp.int32)
    for k in range(NPOS):
        p = hp_ref[0, k]
        acc_c += (p <= col_i).astype(jnp.int32)
        acc_r += (p <= row_i).astype(jnp.int32)
    ridx_ref[...] = acc_c
    onehot_t = (acc_r == lax.broadcasted_iota(jnp.int32, (NSEG, HB), 0)
                ).astype(jnp.float32)
    part = lax.dot_general(onehot_t, x_ref[...], (((1,), (0,)), ((), ())),
                           preferred_element_type=jnp.float32)

    @pl.when(h == 0)
    def _():
        colexp_ref[...] = part

    @pl.when(h > 0)
    def _():
        colexp_ref[...] += part

    @pl.when(h == nsteps - 1)
    def _():
        jj_r = lax.broadcasted_iota(jnp.int32, (1, W), 1)
        jj_c = lax.broadcasted_iota(jnp.int32, (W, 1), 0)
        acc_jr = jnp.zeros((1, W), jnp.int32)
        acc_jc = jnp.zeros((W, 1), jnp.int32)
        for k in range(NPOS):
            p = vp_ref[0, k]
            acc_jr += (p <= jj_r).astype(jnp.int32)
            acc_jc += (p <= jj_c).astype(jnp.int32)
        ohct = (acc_jr == lax.broadcasted_iota(jnp.int32, (NSEG, W), 0)
                ).astype(jnp.float32)
        ohc = (acc_jc == lax.broadcasted_iota(jnp.int32, (W, NSEG), 1)
               ).astype(jnp.float32)
        cnt = jnp.sum(ohc, axis=0, keepdims=True)
        ohc_s = ohc * (1.0 / jnp.maximum(cnt, 1.0))
        for r in range(NSEG):
            lo = hp_ref[0, r - 1] if r > 0 else 0
            hi = hp_ref[0, r] if r < NPOS else H
            rs = 1.0 / jnp.maximum(hi - lo, 1).astype(jnp.float32)
            mean_r = lax.dot_general(
                colexp_ref[r], ohc_s, (((1,), (0,)), ((), ())),
                preferred_element_type=jnp.float32)
            colexp_ref[r] = lax.dot_general(
                mean_r * rs, ohct, (((1,), (0,)), ((), ())),
                preferred_element_type=jnp.float32)


def _row_gather_kernel(ridx_ref, colexp_ref, out_ref):
    base = pl.program_id(0) * IB

    def body(ii, carry):
        r = ridx_ref[base + ii]
        out_ref[pl.ds(ii, 1)] = colexp_ref[pl.ds(r, 1)]
        return carry

    lax.fori_loop(0, IB, body, 0)


def kernel(input, h_positions, v_positions):
    xt = jnp.transpose(input[0], (0, 2, 1))
    hp = h_positions.astype(jnp.int32).reshape(1, NPOS)
    vp = v_positions.astype(jnp.int32).reshape(1, NPOS)

    colexp, ridx = pl.pallas_call(
        _reduce_kernel,
        grid=(H // HB,),
        in_specs=[
            pl.BlockSpec(memory_space=pltpu.SMEM),
            pl.BlockSpec(memory_space=pltpu.SMEM),
            pl.BlockSpec((HB, C, W), lambda h: (h, 0, 0)),
        ],
        out_specs=[
            pl.BlockSpec((NSEG, C, W), lambda h: (0, 0, 0)),
            pl.BlockSpec((HB, 1), lambda h: (h, 0)),
        ],
        out_shape=[
            jax.ShapeDtypeStruct((NSEG, C, W), jnp.float32),
            jax.ShapeDtypeStruct((H, 1), jnp.int32),
        ],
    )(hp, vp, xt)

    yt = pl.pallas_call(
        _row_gather_kernel,
        grid=(H // IB,),
        in_specs=[
            pl.BlockSpec(memory_space=pltpu.SMEM),
            pl.BlockSpec((NSEG, C, W), lambda h: (0, 0, 0)),
        ],
        out_specs=pl.BlockSpec((IB, C, W), lambda h: (h, 0, 0)),
        out_shape=jax.ShapeDtypeStruct((H, C, W), jnp.float32),
    )(ridx.reshape(H), colexp)

    return jnp.transpose(yt, (0, 2, 1))[None]
